# Initial kernel scaffold; baseline (speedup 1.0000x reference)
import jax
import jax.numpy as jnp
from jax.experimental import pallas as pl


def kernel(y, val):
    raise NotImplementedError("write your pallas kernel here")


# speedup vs baseline: 2.6902x; 2.6902x over previous
import functools

import jax
import jax.numpy as jnp
from jax import lax
from jax.experimental import pallas as pl
from jax.experimental.pallas import tpu as pltpu
from jax.experimental.pallas import tpu_sc as plsc

N = 100000
C = 512
NUM_CORES = 2
NUM_SUBCORES = 16
NW = NUM_CORES * NUM_SUBCORES
L = 16

R_FULL = 112
R_TAIL = 96
G_FULL = R_FULL // L
G_TAIL = R_TAIL // L
CHUNK_ELEMS = R_FULL * C
TAIL_ELEMS = R_TAIL * C
NUM_FULL = N // R_FULL
NUM_CHUNKS = NUM_FULL + 1
CHUNKS_PER_W = -(-NUM_CHUNKS // NW)


def _sc_body(y_hbm, val_hbm, zeros_hbm, out_hbm, buf_v, y_v, val_v):
    wid = lax.axis_index("s") * NUM_CORES + lax.axis_index("c")

    pltpu.sync_copy(zeros_hbm, buf_v)
    pltpu.sync_copy(val_hbm, val_v)
    val_vec = val_v[:]
    zero_vec = jnp.zeros((L,), jnp.float32)
    lane = lax.iota(jnp.int32, L)

    def scatter_groups(ngroups, x_vec):
        def g_body(g, _):
            cols = y_v[pl.ds(g * L, L)]
            flat = (lane + g * L) * C + cols
            plsc.store_scatter(buf_v, [flat], x_vec)
            return 0
        lax.fori_loop(0, ngroups, g_body, 0)

    def w_body(i, _):
        cidx = wid + i * NW

        @pl.when(cidx < NUM_FULL)
        def _full():
            pltpu.sync_copy(y_hbm.at[pl.ds(cidx * R_FULL, R_FULL)], y_v)
            scatter_groups(G_FULL, val_vec)
            pltpu.sync_copy(
                buf_v, out_hbm.at[pl.ds(cidx * CHUNK_ELEMS, CHUNK_ELEMS)])
            scatter_groups(G_FULL, zero_vec)

        @pl.when(cidx == NUM_FULL)
        def _tail():
            pltpu.sync_copy(
                y_hbm.at[pl.ds(NUM_FULL * R_FULL, R_TAIL)],
                y_v.at[pl.ds(0, R_TAIL)])
            scatter_groups(G_TAIL, val_vec)
            pltpu.sync_copy(
                buf_v.at[pl.ds(0, TAIL_ELEMS)],
                out_hbm.at[pl.ds(NUM_FULL * CHUNK_ELEMS, TAIL_ELEMS)])
            scatter_groups(G_TAIL, zero_vec)

        return 0

    lax.fori_loop(0, CHUNKS_PER_W, w_body, 0)


_onehot_sc = functools.partial(
    pl.kernel,
    mesh=plsc.VectorSubcoreMesh(core_axis_name="c", subcore_axis_name="s"),
    out_type=jax.ShapeDtypeStruct((N * C,), jnp.float32),
    compiler_params=pltpu.CompilerParams(needs_layout_passes=False),
    scratch_types=[
        pltpu.VMEM((CHUNK_ELEMS,), jnp.float32),
        pltpu.VMEM((R_FULL,), jnp.int32),
        pltpu.VMEM((L,), jnp.float32),
    ],
)(_sc_body)


def kernel(y, val):
    val16 = jnp.broadcast_to(val.astype(jnp.float32), (L,))
    zeros = jnp.zeros((CHUNK_ELEMS,), jnp.float32)
    out_flat = _onehot_sc(y, val16, zeros)
    return out_flat.reshape(N, C)


# speedup vs baseline: 2.7418x; 1.0192x over previous
import functools

import jax
import jax.numpy as jnp
from jax import lax
from jax.experimental import pallas as pl
from jax.experimental.pallas import tpu as pltpu
from jax.experimental.pallas import tpu_sc as plsc

N = 100000
C = 512
NUM_CORES = 2
NUM_SUBCORES = 16
NW = NUM_CORES * NUM_SUBCORES
L = 16

R_FULL = 112
R_TAIL = 96
G_FULL = R_FULL // L
G_TAIL = R_TAIL // L
CHUNK_ELEMS = R_FULL * C
TAIL_ELEMS = R_TAIL * C
NUM_FULL = N // R_FULL
MAX_CH_W = -(-NUM_FULL // NW)
TAIL_WID = NUM_FULL % NW
TAIL_NFULL = (NUM_FULL - TAIL_WID - 1) // NW + 1


def _sc_body(y_hbm, val_hbm, zeros_hbm, out_hbm,
             buf0, buf1, yav, val_v, sem0, sem1, ysem):
    wid = lax.axis_index("s") * NUM_CORES + lax.axis_index("c")
    nfull = jnp.where(wid < NUM_FULL % NW, MAX_CH_W, MAX_CH_W - 1)
    bufs = (buf0, buf1)
    sems = (sem0, sem1)

    pltpu.async_copy(zeros_hbm, buf0, sem0)
    pltpu.async_copy(zeros_hbm, buf1, sem1)

    def y_fire(i, _):
        c = wid + i * NW
        pltpu.async_copy(y_hbm.at[pl.ds(c * R_FULL, R_FULL)],
                         yav.at[pl.ds(i * R_FULL, R_FULL)], ysem)
        return 0
    lax.fori_loop(0, nfull, y_fire, 0)

    pltpu.sync_copy(val_hbm, val_v)
    val_vec = val_v[:]
    zero_vec = jnp.zeros((L,), jnp.float32)
    lane = lax.iota(jnp.int32, L)

    def y_drain(i, _):
        pltpu.make_async_copy(y_hbm.at[pl.ds(0, R_FULL)],
                              yav.at[pl.ds(0, R_FULL)], ysem).wait()
        return 0
    lax.fori_loop(0, nfull, y_drain, 0)

    def scatter_groups(buf, ybase, ngroups, x_vec):
        def g_body(g, _):
            cols = yav[pl.ds(ybase + g * L, L)]
            flat = (lane + g * L) * C + cols
            plsc.store_scatter(buf, [flat], x_vec)
            return 0
        lax.fori_loop(0, ngroups, g_body, 0)

    def process(i, buf, sem):
        c = wid + i * NW
        pltpu.make_async_copy(buf, out_hbm.at[pl.ds(0, CHUNK_ELEMS)],
                              sem).wait()

        @pl.when(i >= 2)
        def _restore():
            scatter_groups(buf, (i - 2) * R_FULL, G_FULL, zero_vec)

        scatter_groups(buf, i * R_FULL, G_FULL, val_vec)
        pltpu.async_copy(
            buf, out_hbm.at[pl.ds(c * CHUNK_ELEMS, CHUNK_ELEMS)], sem)

    def pair_body(p, _):
        for b in range(2):
            i = 2 * p + b

            @pl.when(i < nfull)
            def _():
                process(i, bufs[b], sems[b])
        return 0
    lax.fori_loop(0, (MAX_CH_W + 1) // 2, pair_body, 0)

    for b in range(2):
        pltpu.make_async_copy(bufs[b], out_hbm.at[pl.ds(0, CHUNK_ELEMS)],
                              sems[b]).wait()

    @pl.when(wid == TAIL_WID)
    def _tail():
        scatter_groups(buf0, (TAIL_NFULL - 1) * R_FULL, G_FULL, zero_vec)
        pltpu.sync_copy(y_hbm.at[pl.ds(NUM_FULL * R_FULL, R_TAIL)],
                        yav.at[pl.ds(0, R_TAIL)])
        scatter_groups(buf0, 0, G_TAIL, val_vec)
        pltpu.sync_copy(
            buf0.at[pl.ds(0, TAIL_ELEMS)],
            out_hbm.at[pl.ds(NUM_FULL * CHUNK_ELEMS, TAIL_ELEMS)])


_onehot_sc = functools.partial(
    pl.kernel,
    mesh=plsc.VectorSubcoreMesh(core_axis_name="c", subcore_axis_name="s"),
    out_type=jax.ShapeDtypeStruct((N * C,), jnp.float32),
    compiler_params=pltpu.CompilerParams(needs_layout_passes=False),
    scratch_types=[
        pltpu.VMEM((CHUNK_ELEMS,), jnp.float32),
        pltpu.VMEM((CHUNK_ELEMS,), jnp.float32),
        pltpu.VMEM((MAX_CH_W * R_FULL,), jnp.int32),
        pltpu.VMEM((L,), jnp.float32),
        pltpu.SemaphoreType.DMA,
        pltpu.SemaphoreType.DMA,
        pltpu.SemaphoreType.DMA,
    ],
)(_sc_body)


def kernel(y, val):
    val16 = jnp.broadcast_to(val.astype(jnp.float32), (L,))
    zeros = jnp.zeros((CHUNK_ELEMS,), jnp.float32)
    out_flat = _onehot_sc(y, val16, zeros)
    return out_flat.reshape(N, C)


# speedup vs baseline: 8.2317x; 3.0023x over previous
import functools

import jax
import jax.numpy as jnp
from jax import lax
from jax.experimental import pallas as pl
from jax.experimental.pallas import tpu as pltpu
from jax.experimental.pallas import tpu_sc as plsc

N = 100000
C = 512
NUM_CORES = 2
NUM_SUBCORES = 16
NW = NUM_CORES * NUM_SUBCORES
L = 16

R_FULL = 112
R_TAIL = 96
G_FULL = R_FULL // L
G_TAIL = R_TAIL // L
CHUNK_ELEMS = R_FULL * C
NUM_FULL = N // R_FULL
MAX_CH_W = -(-NUM_FULL // NW)
TAIL_WID = NUM_FULL % NW
TAIL_NFULL = (NUM_FULL - TAIL_WID - 1) // NW + 1


def _sc_body(y_hbm, val_hbm, zeros_hbm, out_hbm,
             buf0, buf1, yav, val_v, sem0, sem1, ysem):
    wid = lax.axis_index("s") * NUM_CORES + lax.axis_index("c")
    nfull = jnp.where(wid < NUM_FULL % NW, MAX_CH_W, MAX_CH_W - 1)
    bufs = (buf0, buf1)
    sems = (sem0, sem1)

    pltpu.async_copy(zeros_hbm, buf0, sem0)
    pltpu.async_copy(zeros_hbm, buf1, sem1)

    def y_fire(i, _):
        c = wid + i * NW
        pltpu.async_copy(y_hbm.at[pl.ds(c * R_FULL, R_FULL)],
                         yav.at[pl.ds(i * R_FULL, R_FULL)], ysem)
        return 0
    lax.fori_loop(0, nfull, y_fire, 0)

    pltpu.sync_copy(val_hbm, val_v)
    val_vec = val_v[:]
    zero_vec = jnp.zeros((L,), jnp.float32)
    lane = lax.iota(jnp.int32, L)

    def y_drain(i, _):
        pltpu.make_async_copy(y_hbm.at[pl.ds(0, R_FULL)],
                              yav.at[pl.ds(0, R_FULL)], ysem).wait()
        return 0
    lax.fori_loop(0, nfull, y_drain, 0)

    def scatter_groups(buf, ybase, ngroups, x_vec):
        def g_body(g, _):
            cols = yav[pl.ds(ybase + g * L, L)]
            rows = lane + g * L
            plsc.store_scatter(buf, [rows, cols], x_vec)
            return 0
        lax.fori_loop(0, ngroups, g_body, 0)

    def process(i, buf, sem):
        c = wid + i * NW
        pltpu.make_async_copy(buf, out_hbm.at[pl.ds(0, R_FULL)], sem).wait()

        @pl.when(i >= 2)
        def _restore():
            scatter_groups(buf, (i - 2) * R_FULL, G_FULL, zero_vec)

        scatter_groups(buf, i * R_FULL, G_FULL, val_vec)
        pltpu.async_copy(buf, out_hbm.at[pl.ds(c * R_FULL, R_FULL)], sem)

    def pair_body(p, _):
        for b in range(2):
            i = 2 * p + b

            @pl.when(i < nfull)
            def _():
                process(i, bufs[b], sems[b])
        return 0
    lax.fori_loop(0, (MAX_CH_W + 1) // 2, pair_body, 0)

    for b in range(2):
        pltpu.make_async_copy(bufs[b], out_hbm.at[pl.ds(0, R_FULL)],
                              sems[b]).wait()

    @pl.when(wid == TAIL_WID)
    def _tail():
        scatter_groups(buf0, (TAIL_NFULL - 1) * R_FULL, G_FULL, zero_vec)
        pltpu.sync_copy(y_hbm.at[pl.ds(NUM_FULL * R_FULL, R_TAIL)],
                        yav.at[pl.ds(0, R_TAIL)])
        scatter_groups(buf0, 0, G_TAIL, val_vec)
        pltpu.sync_copy(
            buf0.at[pl.ds(0, R_TAIL)],
            out_hbm.at[pl.ds(NUM_FULL * R_FULL, R_TAIL)])


_onehot_sc = functools.partial(
    pl.kernel,
    mesh=plsc.VectorSubcoreMesh(core_axis_name="c", subcore_axis_name="s"),
    out_type=jax.ShapeDtypeStruct((N, C), jnp.float32),
    compiler_params=pltpu.CompilerParams(needs_layout_passes=False),
    scratch_types=[
        pltpu.VMEM((R_FULL, C), jnp.float32),
        pltpu.VMEM((R_FULL, C), jnp.float32),
        pltpu.VMEM((MAX_CH_W * R_FULL,), jnp.int32),
        pltpu.VMEM((L,), jnp.float32),
        pltpu.SemaphoreType.DMA,
        pltpu.SemaphoreType.DMA,
        pltpu.SemaphoreType.DMA,
    ],
)(_sc_body)


def kernel(y, val):
    val16 = jnp.broadcast_to(val.astype(jnp.float32), (L,))
    zeros = jnp.zeros((R_FULL, C), jnp.float32)
    return _onehot_sc(y, val16, zeros)


# speedup vs baseline: 8.2439x; 1.0015x over previous
import functools

import jax
import jax.numpy as jnp
from jax import lax
from jax.experimental import pallas as pl
from jax.experimental.pallas import tpu as pltpu
from jax.experimental.pallas import tpu_sc as plsc

N = 100000
C = 512
NUM_CORES = 2
NUM_SUBCORES = 16
NW = NUM_CORES * NUM_SUBCORES
L = 16

R = 112
G = R // L
LAST_ROW0 = N - R
NUM_CHUNKS = -(-N // R)
MAX_CH_W = -(-NUM_CHUNKS // NW)


def _sc_body(y_hbm, val_hbm, zeros_hbm, out_hbm,
             buf0, buf1, yav, val_v, sem0, sem1, ysem):
    wid = lax.axis_index("s") * NUM_CORES + lax.axis_index("c")
    nch = jnp.where(wid < NUM_CHUNKS % NW, MAX_CH_W, MAX_CH_W - 1)
    bufs = (buf0, buf1)
    sems = (sem0, sem1)

    def row0_of(c):
        return pl.multiple_of(jnp.minimum(c * R, LAST_ROW0), 8)

    pltpu.async_copy(zeros_hbm, buf0, sem0)
    pltpu.async_copy(zeros_hbm, buf1, sem1)

    def y_fire(i, _):
        r0 = row0_of(wid + i * NW)
        pltpu.async_copy(y_hbm.at[pl.ds(r0, R)],
                         yav.at[pl.ds(i * R, R)], ysem)
        return 0
    lax.fori_loop(0, nch, y_fire, 0)

    pltpu.sync_copy(val_hbm, val_v)
    val_vec = val_v[:]
    zero_vec = jnp.zeros((L,), jnp.float32)
    lane = lax.iota(jnp.int32, L)

    def y_drain(i, _):
        pltpu.make_async_copy(y_hbm.at[pl.ds(0, R)],
                              yav.at[pl.ds(0, R)], ysem).wait()
        return 0
    lax.fori_loop(0, nch, y_drain, 0)

    def scatter_groups(buf, ybase, x_vec):
        def g_body(g, _):
            cols = yav[pl.ds(ybase + g * L, L)]
            rows = lane + g * L
            plsc.store_scatter(buf, [rows, cols], x_vec)
            return 0
        lax.fori_loop(0, G, g_body, 0)

    def process(i, buf, sem):
        r0 = row0_of(wid + i * NW)
        pltpu.make_async_copy(buf, out_hbm.at[pl.ds(0, R)], sem).wait()

        @pl.when(i >= 2)
        def _restore():
            scatter_groups(buf, (i - 2) * R, zero_vec)

        scatter_groups(buf, i * R, val_vec)
        pltpu.async_copy(buf, out_hbm.at[pl.ds(r0, R)], sem)

    def pair_body(p, _):
        for b in range(2):
            i = 2 * p + b

            @pl.when(i < nch)
            def _():
                process(i, bufs[b], sems[b])
        return 0
    lax.fori_loop(0, (MAX_CH_W + 1) // 2, pair_body, 0)

    for b in range(2):
        pltpu.make_async_copy(bufs[b], out_hbm.at[pl.ds(0, R)],
                              sems[b]).wait()


_onehot_sc = functools.partial(
    pl.kernel,
    mesh=plsc.VectorSubcoreMesh(core_axis_name="c", subcore_axis_name="s"),
    out_type=jax.ShapeDtypeStruct((N, C), jnp.float32),
    compiler_params=pltpu.CompilerParams(needs_layout_passes=False),
    scratch_types=[
        pltpu.VMEM((R, C), jnp.float32),
        pltpu.VMEM((R, C), jnp.float32),
        pltpu.VMEM((MAX_CH_W * R,), jnp.int32),
        pltpu.VMEM((L,), jnp.float32),
        pltpu.SemaphoreType.DMA,
        pltpu.SemaphoreType.DMA,
        pltpu.SemaphoreType.DMA,
    ],
)(_sc_body)


def kernel(y, val):
    val16 = jnp.broadcast_to(val.astype(jnp.float32), (L,))
    zeros = jnp.zeros((R, C), jnp.float32)
    return _onehot_sc(y, val16, zeros)


# speedup vs baseline: 8.3137x; 1.0085x over previous
import functools

import jax
import jax.numpy as jnp
from jax import lax
from jax.experimental import pallas as pl
from jax.experimental.pallas import tpu as pltpu
from jax.experimental.pallas import tpu_sc as plsc

N = 100000
C = 512
NUM_CORES = 2
NUM_SUBCORES = 16
NW = NUM_CORES * NUM_SUBCORES
L = 16

R = 112
G = R // L
LAST_ROW0 = N - R
NUM_CHUNKS = -(-N // R)
MAX_CH_W = -(-NUM_CHUNKS // NW)


def _sc_body(y_hbm, val_hbm, zeros_hbm, out_hbm,
             buf0, buf1, yav, val_v, sem0, sem1, ysem):
    wid = lax.axis_index("s") * NUM_CORES + lax.axis_index("c")
    nch = jnp.where(wid < NUM_CHUNKS % NW, MAX_CH_W, MAX_CH_W - 1)
    bufs = (buf0, buf1)
    sems = (sem0, sem1)

    def row0_of(c):
        return pl.multiple_of(jnp.minimum(c * R, LAST_ROW0), 8)

    pltpu.async_copy(zeros_hbm, buf0, sem0)
    pltpu.async_copy(zeros_hbm, buf1, sem1)

    def y_fire(i, _):
        r0 = row0_of(wid + i * NW)
        pltpu.async_copy(y_hbm.at[pl.ds(r0, R)],
                         yav.at[pl.ds(i * R, R)], ysem)
        return 0
    lax.fori_loop(0, nch, y_fire, 0)

    pltpu.sync_copy(val_hbm, val_v)
    val_vec = val_v[:]
    zero_vec = jnp.zeros((L,), jnp.float32)
    lane = lax.iota(jnp.int32, L)

    def y_drain(i, _):
        pltpu.make_async_copy(y_hbm.at[pl.ds(0, R)],
                              yav.at[pl.ds(0, R)], ysem).wait()
        return 0
    lax.fori_loop(0, nch, y_drain, 0)

    def scatter_groups(buf, ybase, x_vec):
        def g_body(g, _):
            cols = yav[pl.ds(ybase + g * L, L)]
            rows = lane + g * L
            plsc.store_scatter(buf, [rows, cols], x_vec)
            return 0
        lax.fori_loop(0, G, g_body, 0)

    def process(i, buf, sem):
        r0 = row0_of(wid + i * NW)
        pltpu.make_async_copy(buf, out_hbm.at[pl.ds(0, R)], sem).wait()

        @pl.when(i >= 2)
        def _restore():
            scatter_groups(buf, (i - 2) * R, zero_vec)

        scatter_groups(buf, i * R, val_vec)
        pltpu.async_copy(buf, out_hbm.at[pl.ds(r0, R)], sem)

    def pair_body(p, _):
        for b in range(2):
            i = 2 * p + b

            @pl.when(i < nch)
            def _():
                process(i, bufs[b], sems[b])
        return 0
    lax.fori_loop(0, (MAX_CH_W + 1) // 2, pair_body, 0)

    for b in range(2):
        pltpu.make_async_copy(bufs[b], out_hbm.at[pl.ds(0, R)],
                              sems[b]).wait()


_onehot_sc = functools.partial(
    pl.kernel,
    mesh=plsc.VectorSubcoreMesh(core_axis_name="c", subcore_axis_name="s"),
    out_type=jax.ShapeDtypeStruct((N, C), jnp.float32),
    compiler_params=pltpu.CompilerParams(
        needs_layout_passes=False, skip_device_barrier=True),
    scratch_types=[
        pltpu.VMEM((R, C), jnp.float32),
        pltpu.VMEM((R, C), jnp.float32),
        pltpu.VMEM((MAX_CH_W * R,), jnp.int32),
        pltpu.VMEM((L,), jnp.float32),
        pltpu.SemaphoreType.DMA,
        pltpu.SemaphoreType.DMA,
        pltpu.SemaphoreType.DMA,
    ],
)(_sc_body)


def kernel(y, val):
    val16 = jnp.broadcast_to(val.astype(jnp.float32), (L,))
    zeros = jnp.zeros((R, C), jnp.float32)
    return _onehot_sc(y, val16, zeros)
